# baseline (speedup 1.0000x reference)
import functools

import jax
import jax.numpy as jnp
from jax import lax
from jax.experimental import pallas as pl
from jax.experimental.pallas import tpu as pltpu

_CBH = 2
_NBUF = 6


def _body(S, U, BH, pos_ref, cache_any, upd_any, out_any,
          in_bufs, out_bufs, upd_v, in_sems, out_sems, usem):
    nchunk = BH // _CBH
    upd_cp = pltpu.make_async_copy(upd_any, upd_v, usem)
    upd_cp.start()

    def in_copy(k):
        return pltpu.make_async_copy(
            cache_any.at[pl.ds(k * _CBH, _CBH)],
            in_bufs.at[k % _NBUF],
            in_sems.at[k % _NBUF],
        )

    def out_copy(k):
        return pltpu.make_async_copy(
            out_bufs.at[k % _NBUF],
            out_any.at[pl.ds(k * _CBH, _CBH)],
            out_sems.at[k % _NBUF],
        )

    for k in range(_NBUF):
        in_copy(k).start()
    upd_cp.wait()
    p = pos_ref[0]

    for k in range(nchunk):
        b = k % _NBUF
        if k >= _NBUF:
            out_copy(k - _NBUF).wait()
        in_copy(k).wait()
        out_bufs[b] = in_bufs[b]
        for j in range(_CBH):
            bh = k * _CBH + j
            for i in range(U):
                r = lax.rem(p + i, S)
                out_bufs[b, j, pl.ds(r, 1), :] = upd_v[bh, pl.ds(i, 1), :]
        out_copy(k).start()
        if k + _NBUF < nchunk:
            in_copy(k + _NBUF).start()
    for k in range(nchunk - _NBUF, nchunk):
        out_copy(k).wait()


def kernel(cache, update, pos):
    B, H, S, D = cache.shape
    U = update.shape[-2]
    BH = B * H
    cache3 = cache.reshape(BH, S, D)
    update3 = update.reshape(BH, U, D)
    pos_arr = jnp.asarray(pos, jnp.int32).reshape(1)

    out = pl.pallas_call(
        functools.partial(_body, S, U, BH),
        out_shape=jax.ShapeDtypeStruct((BH, S, D), cache.dtype),
        in_specs=[
            pl.BlockSpec(memory_space=pltpu.SMEM),
            pl.BlockSpec(memory_space=pl.ANY),
            pl.BlockSpec(memory_space=pl.ANY),
        ],
        out_specs=pl.BlockSpec(memory_space=pl.ANY),
        scratch_shapes=[
            pltpu.VMEM((_NBUF, _CBH, S, D), jnp.float32),
            pltpu.VMEM((_NBUF, _CBH, S, D), jnp.float32),
            pltpu.VMEM((BH, U, D), jnp.float32),
            pltpu.SemaphoreType.DMA((_NBUF,)),
            pltpu.SemaphoreType.DMA((_NBUF,)),
            pltpu.SemaphoreType.DMA,
        ],
        compiler_params=pltpu.CompilerParams(
            vmem_limit_bytes=64 * 1024 * 1024,
        ),
        name="kvcache_ring_copy_update",
    )(pos_arr, cache3, update3)
    return out.reshape(B, H, S, D)


# speedup vs baseline: 1.0014x; 1.0014x over previous
import functools

import jax
import jax.numpy as jnp
from jax import lax
from jax.experimental import pallas as pl
from jax.experimental.pallas import tpu as pltpu

_CBH = 4
_NBI = 4
_NBO = 3


def _body(S, U, BH, pos_ref, cache_any, upd_any, out_any,
          in_bufs, out_bufs, upd_v, in_sems, out_sems, usem):
    nchunk = BH // _CBH
    upd_cp = pltpu.make_async_copy(upd_any, upd_v, usem)
    upd_cp.start()

    def in_copy(k):
        return pltpu.make_async_copy(
            cache_any.at[pl.ds(k * _CBH, _CBH)],
            in_bufs.at[k % _NBI],
            in_sems.at[k % _NBI],
        )

    def out_copy(k):
        return pltpu.make_async_copy(
            out_bufs.at[k % _NBO],
            out_any.at[pl.ds(k * _CBH, _CBH)],
            out_sems.at[k % _NBO],
        )

    for k in range(_NBI):
        in_copy(k).start()
    upd_cp.wait()
    p = pos_ref[0]

    for k in range(nchunk):
        bi = k % _NBI
        bo = k % _NBO
        if k >= _NBO:
            out_copy(k - _NBO).wait()
        in_copy(k).wait()
        out_bufs[bo] = in_bufs[bi]
        for j in range(_CBH):
            bh = k * _CBH + j
            for i in range(U):
                r = lax.rem(p + i, S)
                out_bufs[bo, j, pl.ds(r, 1), :] = upd_v[bh, pl.ds(i, 1), :]
        out_copy(k).start()
        if k + _NBI < nchunk:
            in_copy(k + _NBI).start()
    for k in range(nchunk - _NBO, nchunk):
        out_copy(k).wait()


def kernel(cache, update, pos):
    B, H, S, D = cache.shape
    U = update.shape[-2]
    BH = B * H
    cache3 = cache.reshape(BH, S, D)
    update3 = update.reshape(BH, U, D)
    pos_arr = jnp.asarray(pos, jnp.int32).reshape(1)

    out = pl.pallas_call(
        functools.partial(_body, S, U, BH),
        out_shape=jax.ShapeDtypeStruct((BH, S, D), cache.dtype),
        in_specs=[
            pl.BlockSpec(memory_space=pltpu.SMEM),
            pl.BlockSpec(memory_space=pl.ANY),
            pl.BlockSpec(memory_space=pl.ANY),
        ],
        out_specs=pl.BlockSpec(memory_space=pl.ANY),
        scratch_shapes=[
            pltpu.VMEM((_NBI, _CBH, S, D), jnp.float32),
            pltpu.VMEM((_NBO, _CBH, S, D), jnp.float32),
            pltpu.VMEM((BH, U, D), jnp.float32),
            pltpu.SemaphoreType.DMA((_NBI,)),
            pltpu.SemaphoreType.DMA((_NBO,)),
            pltpu.SemaphoreType.DMA,
        ],
        compiler_params=pltpu.CompilerParams(
            vmem_limit_bytes=64 * 1024 * 1024,
        ),
        name="kvcache_ring_copy_update",
    )(pos_arr, cache3, update3)
    return out.reshape(B, H, S, D)
